# baseline (speedup 1.0000x reference)
import functools

import jax
import jax.numpy as jnp
from jax import lax
from jax.experimental import pallas as pl
from jax.experimental.pallas import tpu as pltpu
from jax.experimental.pallas import tpu_sc as plsc

N_NODES = 10000
N_EDGES = 320000
D_FEAT = 128
BATCH_SIZE = 100
NODES_PER_GRAPH = 100

NC = 2
NS = 16
NW = NC * NS
CHUNK = 80
NCHUNKS = 128
EPW = NCHUNKS * CHUNK
E_PAD = NW * EPW
N_PAD = 10240
RPT = N_PAD // NS
RCH = RPT // CHUNK


def _sc_agg_body(with_deg, *refs):
    if with_deg:
        (h_hbm, src_hbm, dst_hbm, zfeat_hbm, zdeg_hbm, ones_hbm,
         agg_out, deg_out,
         agg_sh, deg_sh, r0, r1, r2, r3, s0, s1, s2, s3, d0, d1, d2, d3,
         ones_v, dstage_v,
         g0, g1, g2, g3, si0, si1, si2, si3, di0, di1, di2, di3) = refs
    else:
        (h_hbm, src_hbm, dst_hbm, zfeat_hbm,
         agg_out,
         agg_sh, r0, r1, r2, r3, s0, s1, s2, s3, d0, d1, d2, d3,
         g0, g1, g2, g3, si0, si1, si2, si3, di0, di1, di2, di3) = refs

    rows = (r0, r1, r2, r3)
    sidx = (s0, s1, s2, s3)
    didx = (d0, d1, d2, d3)
    gsem = (g0, g1, g2, g3)
    ssem = (si0, si1, si2, si3)
    dsem = (di0, di1, di2, di3)

    cid = lax.axis_index("c")
    sid = lax.axis_index("s")
    wid = sid * NC + cid

    pltpu.sync_copy(zfeat_hbm, r0)
    for k in range(RCH):
        pltpu.sync_copy(r0, agg_sh.at[pl.ds(sid * RPT + k * CHUNK, CHUNK)])
    if with_deg:
        pltpu.sync_copy(zdeg_hbm, dstage_v)
        pltpu.sync_copy(dstage_v, deg_sh.at[pl.ds(sid * RPT, RPT)])
        pltpu.sync_copy(ones_hbm, ones_v)
    plsc.subcore_barrier()

    def sfetch(i, m):
        pltpu.async_copy(
            src_hbm.at[pl.ds(wid * EPW + i * CHUNK, CHUNK)], sidx[m], ssem[m])

    def swait(m):
        pltpu.make_async_copy(
            src_hbm.at[pl.ds(0, CHUNK)], sidx[m], ssem[m]).wait()

    def dfetch(i, m):
        pltpu.async_copy(
            dst_hbm.at[pl.ds(wid * EPW + i * CHUNK, CHUNK)], didx[m], dsem[m])

    def dwait(m):
        pltpu.make_async_copy(
            dst_hbm.at[pl.ds(0, CHUNK)], didx[m], dsem[m]).wait()

    def gather(m):
        pltpu.async_copy(h_hbm.at[sidx[m]], rows[m], gsem[m])

    def gwait(m):
        pltpu.make_async_copy(h_hbm.at[sidx[0]], rows[m], gsem[m]).wait()

    def scatter(m):
        pltpu.sync_copy(rows[m], agg_sh.at[didx[m]], add=True)
        if with_deg:
            pltpu.sync_copy(ones_v, deg_sh.at[didx[m]], add=True)

    for m in range(4):
        sfetch(m, m)
        dfetch(m, m)
    swait(0)
    gather(0)
    swait(1)
    gather(1)

    def group(g, carry):
        j0 = 4 * g
        for m in range(4):
            j = j0 + m
            gwait(m)
            sfetch(j + 4, m)
            swait((m + 2) % 4)
            gather((m + 2) % 4)
            dwait(m)
            scatter(m)
            dfetch(j + 4, m)
        return carry

    lax.fori_loop(0, (NCHUNKS - 4) // 4, group, 0)
    gwait(0)
    swait(2)
    gather(2)
    dwait(0)
    scatter(0)
    gwait(1)
    swait(3)
    gather(3)
    dwait(1)
    scatter(1)
    gwait(2)
    dwait(2)
    scatter(2)
    gwait(3)
    dwait(3)
    scatter(3)
    plsc.subcore_barrier()

    for k in range(RCH):
        pltpu.sync_copy(agg_sh.at[pl.ds(sid * RPT + k * CHUNK, CHUNK)], r0)
        pltpu.sync_copy(r0, agg_out.at[cid, pl.ds(sid * RPT + k * CHUNK, CHUNK)])
    if with_deg:
        pltpu.sync_copy(deg_sh.at[pl.ds(sid * RPT, RPT)], dstage_v)
        pltpu.sync_copy(dstage_v, deg_out.at[pl.ds(cid * N_PAD + sid * RPT, RPT)])


def _sc_aggregate(h, src, dst, with_deg):
    mesh = plsc.VectorSubcoreMesh(core_axis_name="c", subcore_axis_name="s",
                                  num_cores=NC, num_subcores=NS)
    zfeat = jnp.zeros((CHUNK, D_FEAT), jnp.float32)
    rows_t = pltpu.VMEM((CHUNK, D_FEAT), jnp.float32)
    idx_t = pltpu.VMEM((CHUNK,), jnp.int32)
    sem = pltpu.SemaphoreType.DMA
    ring = [rows_t] * 4 + [idx_t] * 8
    sems = [sem] * 12
    if with_deg:
        out_type = (jax.ShapeDtypeStruct((NC, N_PAD, D_FEAT), jnp.float32),
                    jax.ShapeDtypeStruct((NC * N_PAD,), jnp.float32))
        scratch = [
            pltpu.VMEM_SHARED((N_PAD, D_FEAT), jnp.float32),
            pltpu.VMEM_SHARED((N_PAD,), jnp.float32),
        ] + ring + [
            pltpu.VMEM((CHUNK,), jnp.float32),
            pltpu.VMEM((RPT,), jnp.float32),
        ] + sems
        zdeg = jnp.zeros((RPT,), jnp.float32)
        ones = jnp.ones((CHUNK,), jnp.float32)
        fn = pl.kernel(functools.partial(_sc_agg_body, True),
                       out_type=out_type, mesh=mesh, scratch_types=scratch)
        return fn(h, src, dst, zfeat, zdeg, ones)
    else:
        out_type = jax.ShapeDtypeStruct((NC, N_PAD, D_FEAT), jnp.float32)
        scratch = [
            pltpu.VMEM_SHARED((N_PAD, D_FEAT), jnp.float32),
        ] + ring + sems
        fn = pl.kernel(functools.partial(_sc_agg_body, False),
                       out_type=out_type, mesh=mesh, scratch_types=scratch)
        return fn(h, src, dst, zfeat)


def _dense1_body(x_ref, agg_ref, degA_ref, degB_ref, Wr_ref, Wa_ref, b_ref, g_ref,
                 be_ref, o_ref):
    agg = agg_ref[0, :N_NODES] + agg_ref[1, :N_NODES]
    deg = degA_ref[:N_NODES] + degB_ref[:N_NODES]
    mean = agg / jnp.maximum(deg, 1.0)
    c = (jnp.dot(x_ref[...], Wr_ref[...], preferred_element_type=jnp.float32)
         + jnp.dot(mean, Wa_ref[...], preferred_element_type=jnp.float32)
         + b_ref[...])
    mu = jnp.mean(c, axis=0, keepdims=True)
    var = jnp.mean((c - mu) * (c - mu), axis=0, keepdims=True)
    h = (c - mu) * lax.rsqrt(var + 1e-5) * g_ref[...] + be_ref[...]
    o_ref[...] = jnp.maximum(h, 0.0)


def _dense1(x, agg, degA, degB, Wr, Wa, b, g, be):
    return pl.pallas_call(
        _dense1_body,
        out_shape=jax.ShapeDtypeStruct((N_NODES, D_FEAT), jnp.float32),
    )(x, agg, degA, degB, Wr, Wa, b.reshape(1, -1),
      g.reshape(1, -1), be.reshape(1, -1))


def _dense2_body(h_ref, agg_ref, degA_ref, degB_ref, Wr_ref, Wa_ref, b_ref, g_ref,
                 be_ref, Ws1_ref, bs1_ref, Ws2_ref, bs2_ref, Wh1_ref,
                 bh1_ref, Wh2_ref, bh2_ref, Wh3_ref, bh3_ref, o_ref):
    agg = agg_ref[0, :N_NODES] + agg_ref[1, :N_NODES]
    deg = degA_ref[:N_NODES] + degB_ref[:N_NODES]
    mean = agg / jnp.maximum(deg, 1.0)
    c = (jnp.dot(h_ref[...], Wr_ref[...], preferred_element_type=jnp.float32)
         + jnp.dot(mean, Wa_ref[...], preferred_element_type=jnp.float32)
         + b_ref[...])
    mu = jnp.mean(c, axis=0, keepdims=True)
    var = jnp.mean((c - mu) * (c - mu), axis=0, keepdims=True)
    h = (c - mu) * lax.rsqrt(var + 1e-5) * g_ref[...] + be_ref[...]
    h = jnp.maximum(h, 0.0)

    hg = jnp.mean(h.reshape(BATCH_SIZE, NODES_PER_GRAPH, D_FEAT), axis=1)

    t = jnp.maximum(hg, 0.0)
    t = jnp.dot(t, Ws1_ref[...], preferred_element_type=jnp.float32) + bs1_ref[...]
    t = jnp.dot(t, Ws2_ref[...], preferred_element_type=jnp.float32) + bs2_ref[...]
    t = jnp.maximum(t, 0.0)
    t = jnp.maximum(jnp.dot(t, Wh1_ref[...], preferred_element_type=jnp.float32) + bh1_ref[...], 0.0)
    t = jnp.maximum(jnp.dot(t, Wh2_ref[...], preferred_element_type=jnp.float32) + bh2_ref[...], 0.0)
    o_ref[...] = jnp.dot(t, Wh3_ref[...], preferred_element_type=jnp.float32) + bh3_ref[...]


def _dense2(h, agg, degA, degB, Wr, Wa, b, g, be, Ws1, bs1, Ws2, bs2,
            Wh1, bh1, Wh2, bh2, Wh3, bh3):
    return pl.pallas_call(
        _dense2_body,
        out_shape=jax.ShapeDtypeStruct((BATCH_SIZE, 1), jnp.float32),
    )(h, agg, degA, degB, Wr, Wa, b.reshape(1, -1), g.reshape(1, -1),
      be.reshape(1, -1), Ws1, bs1.reshape(1, -1), Ws2, bs2.reshape(1, -1),
      Wh1, bh1.reshape(1, -1), Wh2, bh2.reshape(1, -1), Wh3,
      bh3.reshape(1, -1))


def kernel(x, edge_index, batch, Wr0, Wa0, b0, g0, be0, Wr1, Wa1, b1, g1,
           be1, Ws1, bs1, Ws2, bs2, Wh1, bh1, Wh2, bh2, Wh3, bh3):
    npad = E_PAD - N_EDGES
    src = jnp.concatenate([edge_index[0], jnp.zeros((npad,), jnp.int32)])
    dst = jnp.concatenate(
        [edge_index[1], jnp.full((npad,), N_PAD - 1, jnp.int32)])
    agg0, deg_flat = _sc_aggregate(x, src, dst, with_deg=True)
    degA = deg_flat[:N_PAD].reshape(N_PAD, 1)
    degB = deg_flat[N_PAD:].reshape(N_PAD, 1)
    h1 = _dense1(x, agg0, degA, degB, Wr0, Wa0, b0, g0, be0)
    agg1 = _sc_aggregate(h1, src, dst, with_deg=False)
    return _dense2(h1, agg1, degA, degB, Wr1, Wa1, b1, g1, be1, Ws1, bs1,
                   Ws2, bs2, Wh1, bh1, Wh2, bh2, Wh3, bh3)


# speedup vs baseline: 1.1487x; 1.1487x over previous
import functools

import jax
import jax.numpy as jnp
from jax import lax
from jax.experimental import pallas as pl
from jax.experimental.pallas import tpu as pltpu
from jax.experimental.pallas import tpu_sc as plsc

N_NODES = 10000
N_EDGES = 320000
D_FEAT = 128
BATCH_SIZE = 100
NODES_PER_GRAPH = 100

NC = 2
NS = 16
NW = NC * NS
CHUNK = 64
NCHUNKS = 160
EPW = NCHUNKS * CHUNK
E_PAD = NW * EPW
N_PAD = 10240
RPT = N_PAD // NS
RCH = RPT // CHUNK


def _sc_agg_body(with_deg, *refs):
    if with_deg:
        (h_hbm, src_hbm, dst_hbm, zfeat_hbm, zdeg_hbm, ones_hbm,
         agg_out, deg_out,
         agg_sh, deg_sh, rows0_v, rows1_v, rows2_v, sidx_v, didx_v,
         ones_v, dstage_v, sem0, sem1, sem2) = refs
    else:
        (h_hbm, src_hbm, dst_hbm, zfeat_hbm,
         agg_out,
         agg_sh, rows0_v, rows1_v, rows2_v, sidx_v, didx_v,
         sem0, sem1, sem2) = refs
    rows = (rows0_v, rows1_v, rows2_v)
    sems = (sem0, sem1, sem2)

    cid = lax.axis_index("c")
    sid = lax.axis_index("s")
    wid = sid * NC + cid

    pltpu.sync_copy(src_hbm.at[pl.ds(wid * EPW, EPW)], sidx_v)
    pltpu.sync_copy(dst_hbm.at[pl.ds(wid * EPW, EPW)], didx_v)

    pltpu.sync_copy(zfeat_hbm, rows0_v)
    for k in range(RCH):
        pltpu.sync_copy(rows0_v, agg_sh.at[pl.ds(sid * RPT + k * CHUNK, CHUNK)])
    if with_deg:
        pltpu.sync_copy(zdeg_hbm, dstage_v)
        pltpu.sync_copy(dstage_v, deg_sh.at[pl.ds(sid * RPT, RPT)])
        pltpu.sync_copy(ones_hbm, ones_v)
    plsc.subcore_barrier()

    def gather(i, k):
        pltpu.async_copy(
            h_hbm.at[sidx_v.at[pl.ds(i * CHUNK, CHUNK)]], rows[k], sems[k])

    def gwait(k):
        pltpu.make_async_copy(
            h_hbm.at[sidx_v.at[pl.ds(0, CHUNK)]], rows[k], sems[k]).wait()

    def scatter(i, k):
        idx = didx_v.at[pl.ds(i * CHUNK, CHUNK)]
        pltpu.sync_copy(rows[k], agg_sh.at[idx], add=True)
        if with_deg:
            pltpu.sync_copy(ones_v, deg_sh.at[idx], add=True)

    gather(0, 0)
    gather(1, 1)

    def group(g, carry):
        j0 = 3 * g
        for c in range(3):
            j = j0 + c
            gwait(c)
            gather(j + 2, (c + 2) % 3)
            scatter(j, c)
        return carry

    lax.fori_loop(0, (NCHUNKS - 4) // 3, group, 0)
    gwait(0)
    gather(NCHUNKS - 2, 2)
    scatter(NCHUNKS - 4, 0)
    gwait(1)
    gather(NCHUNKS - 1, 0)
    scatter(NCHUNKS - 3, 1)
    gwait(2)
    scatter(NCHUNKS - 2, 2)
    gwait(0)
    scatter(NCHUNKS - 1, 0)
    plsc.subcore_barrier()

    for k in range(RCH):
        pltpu.sync_copy(agg_sh.at[pl.ds(sid * RPT + k * CHUNK, CHUNK)], rows0_v)
        pltpu.sync_copy(rows0_v, agg_out.at[cid, pl.ds(sid * RPT + k * CHUNK, CHUNK)])
    if with_deg:
        pltpu.sync_copy(deg_sh.at[pl.ds(sid * RPT, RPT)], dstage_v)
        pltpu.sync_copy(dstage_v, deg_out.at[pl.ds(cid * N_PAD + sid * RPT, RPT)])


def _sc_aggregate(h, src, dst, with_deg):
    mesh = plsc.VectorSubcoreMesh(core_axis_name="c", subcore_axis_name="s",
                                  num_cores=NC, num_subcores=NS)
    zfeat = jnp.zeros((CHUNK, D_FEAT), jnp.float32)
    if with_deg:
        out_type = (jax.ShapeDtypeStruct((NC, N_PAD, D_FEAT), jnp.float32),
                    jax.ShapeDtypeStruct((NC * N_PAD,), jnp.float32))
        scratch = [
            pltpu.VMEM_SHARED((N_PAD, D_FEAT), jnp.float32),
            pltpu.VMEM_SHARED((N_PAD,), jnp.float32),
            pltpu.VMEM((CHUNK, D_FEAT), jnp.float32),
            pltpu.VMEM((CHUNK, D_FEAT), jnp.float32),
            pltpu.VMEM((CHUNK, D_FEAT), jnp.float32),
            pltpu.VMEM((EPW,), jnp.int32),
            pltpu.VMEM((EPW,), jnp.int32),
            pltpu.VMEM((CHUNK,), jnp.float32),
            pltpu.VMEM((RPT,), jnp.float32),
            pltpu.SemaphoreType.DMA,
            pltpu.SemaphoreType.DMA,
            pltpu.SemaphoreType.DMA,
        ]
        zdeg = jnp.zeros((RPT,), jnp.float32)
        ones = jnp.ones((CHUNK,), jnp.float32)
        fn = pl.kernel(functools.partial(_sc_agg_body, True),
                       out_type=out_type, mesh=mesh, scratch_types=scratch)
        return fn(h, src, dst, zfeat, zdeg, ones)
    else:
        out_type = jax.ShapeDtypeStruct((NC, N_PAD, D_FEAT), jnp.float32)
        scratch = [
            pltpu.VMEM_SHARED((N_PAD, D_FEAT), jnp.float32),
            pltpu.VMEM((CHUNK, D_FEAT), jnp.float32),
            pltpu.VMEM((CHUNK, D_FEAT), jnp.float32),
            pltpu.VMEM((CHUNK, D_FEAT), jnp.float32),
            pltpu.VMEM((EPW,), jnp.int32),
            pltpu.VMEM((EPW,), jnp.int32),
            pltpu.SemaphoreType.DMA,
            pltpu.SemaphoreType.DMA,
            pltpu.SemaphoreType.DMA,
        ]
        fn = pl.kernel(functools.partial(_sc_agg_body, False),
                       out_type=out_type, mesh=mesh, scratch_types=scratch)
        return fn(h, src, dst, zfeat)


def _dense1_body(x_ref, agg_ref, degA_ref, degB_ref, Wr_ref, Wa_ref, b_ref, g_ref,
                 be_ref, o_ref):
    agg = agg_ref[0, :N_NODES] + agg_ref[1, :N_NODES]
    deg = degA_ref[:N_NODES] + degB_ref[:N_NODES]
    mean = agg / jnp.maximum(deg, 1.0)
    c = (jnp.dot(x_ref[...], Wr_ref[...], preferred_element_type=jnp.float32)
         + jnp.dot(mean, Wa_ref[...], preferred_element_type=jnp.float32)
         + b_ref[...])
    mu = jnp.mean(c, axis=0, keepdims=True)
    var = jnp.mean((c - mu) * (c - mu), axis=0, keepdims=True)
    h = (c - mu) * lax.rsqrt(var + 1e-5) * g_ref[...] + be_ref[...]
    o_ref[...] = jnp.maximum(h, 0.0)


def _dense1(x, agg, degA, degB, Wr, Wa, b, g, be):
    return pl.pallas_call(
        _dense1_body,
        out_shape=jax.ShapeDtypeStruct((N_NODES, D_FEAT), jnp.float32),
    )(x, agg, degA, degB, Wr, Wa, b.reshape(1, -1),
      g.reshape(1, -1), be.reshape(1, -1))


def _dense2_body(h_ref, agg_ref, degA_ref, degB_ref, Wr_ref, Wa_ref, b_ref, g_ref,
                 be_ref, Ws1_ref, bs1_ref, Ws2_ref, bs2_ref, Wh1_ref,
                 bh1_ref, Wh2_ref, bh2_ref, Wh3_ref, bh3_ref, o_ref):
    agg = agg_ref[0, :N_NODES] + agg_ref[1, :N_NODES]
    deg = degA_ref[:N_NODES] + degB_ref[:N_NODES]
    mean = agg / jnp.maximum(deg, 1.0)
    c = (jnp.dot(h_ref[...], Wr_ref[...], preferred_element_type=jnp.float32)
         + jnp.dot(mean, Wa_ref[...], preferred_element_type=jnp.float32)
         + b_ref[...])
    mu = jnp.mean(c, axis=0, keepdims=True)
    var = jnp.mean((c - mu) * (c - mu), axis=0, keepdims=True)
    h = (c - mu) * lax.rsqrt(var + 1e-5) * g_ref[...] + be_ref[...]
    h = jnp.maximum(h, 0.0)

    hg = jnp.mean(h.reshape(BATCH_SIZE, NODES_PER_GRAPH, D_FEAT), axis=1)

    t = jnp.maximum(hg, 0.0)
    t = jnp.dot(t, Ws1_ref[...], preferred_element_type=jnp.float32) + bs1_ref[...]
    t = jnp.dot(t, Ws2_ref[...], preferred_element_type=jnp.float32) + bs2_ref[...]
    t = jnp.maximum(t, 0.0)
    t = jnp.maximum(jnp.dot(t, Wh1_ref[...], preferred_element_type=jnp.float32) + bh1_ref[...], 0.0)
    t = jnp.maximum(jnp.dot(t, Wh2_ref[...], preferred_element_type=jnp.float32) + bh2_ref[...], 0.0)
    o_ref[...] = jnp.dot(t, Wh3_ref[...], preferred_element_type=jnp.float32) + bh3_ref[...]


def _dense2(h, agg, degA, degB, Wr, Wa, b, g, be, Ws1, bs1, Ws2, bs2,
            Wh1, bh1, Wh2, bh2, Wh3, bh3):
    return pl.pallas_call(
        _dense2_body,
        out_shape=jax.ShapeDtypeStruct((BATCH_SIZE, 1), jnp.float32),
    )(h, agg, degA, degB, Wr, Wa, b.reshape(1, -1), g.reshape(1, -1),
      be.reshape(1, -1), Ws1, bs1.reshape(1, -1), Ws2, bs2.reshape(1, -1),
      Wh1, bh1.reshape(1, -1), Wh2, bh2.reshape(1, -1), Wh3,
      bh3.reshape(1, -1))


def kernel(x, edge_index, batch, Wr0, Wa0, b0, g0, be0, Wr1, Wa1, b1, g1,
           be1, Ws1, bs1, Ws2, bs2, Wh1, bh1, Wh2, bh2, Wh3, bh3):
    npad = E_PAD - N_EDGES
    src = jnp.concatenate([edge_index[0], jnp.zeros((npad,), jnp.int32)])
    dst = jnp.concatenate(
        [edge_index[1], jnp.full((npad,), N_PAD - 1, jnp.int32)])
    agg0, deg_flat = _sc_aggregate(x, src, dst, with_deg=True)
    degA = deg_flat[:N_PAD].reshape(N_PAD, 1)
    degB = deg_flat[N_PAD:].reshape(N_PAD, 1)
    h1 = _dense1(x, agg0, degA, degB, Wr0, Wa0, b0, g0, be0)
    agg1 = _sc_aggregate(h1, src, dst, with_deg=False)
    return _dense2(h1, agg1, degA, degB, Wr1, Wa1, b1, g1, be1, Ws1, bs1,
                   Ws2, bs2, Wh1, bh1, Wh2, bh2, Wh3, bh3)


# speedup vs baseline: 1.1490x; 1.0003x over previous
import functools

import jax
import jax.numpy as jnp
from jax import lax
from jax.experimental import pallas as pl
from jax.experimental.pallas import tpu as pltpu
from jax.experimental.pallas import tpu_sc as plsc

N_NODES = 10000
N_EDGES = 320000
D_FEAT = 128
BATCH_SIZE = 100
NODES_PER_GRAPH = 100

NC = 2
NS = 16
NW = NC * NS
CHUNK = 64
NCHUNKS = 160
EPW = NCHUNKS * CHUNK
E_PAD = NW * EPW
N_PAD = 10240
RPT = N_PAD // NS
RCH = RPT // CHUNK


def _sc_agg_body(with_deg, *refs):
    if with_deg:
        (h_hbm, src_hbm, dst_hbm, zfeat_hbm, zdeg_hbm, ones_hbm,
         agg_out, deg_out,
         agg_sh, deg_sh, rows0_v, rows1_v, rows2_v, sidx_v, didx_v,
         ones_v, dstage_v, sem0, sem1, sem2) = refs
    else:
        (h_hbm, src_hbm, dst_hbm, zfeat_hbm,
         agg_out,
         agg_sh, rows0_v, rows1_v, rows2_v, sidx_v, didx_v,
         sem0, sem1, sem2) = refs
    rows = (rows0_v, rows1_v, rows2_v)
    sems = (sem0, sem1, sem2)

    cid = lax.axis_index("c")
    sid = lax.axis_index("s")
    wid = sid * NC + cid

    pltpu.sync_copy(src_hbm.at[pl.ds(wid * EPW, EPW)], sidx_v)
    pltpu.sync_copy(dst_hbm.at[pl.ds(wid * EPW, EPW)], didx_v)

    pltpu.sync_copy(zfeat_hbm, rows0_v)
    for k in range(RCH):
        pltpu.sync_copy(rows0_v, agg_sh.at[pl.ds(sid * RPT + k * CHUNK, CHUNK)])
    if with_deg:
        pltpu.sync_copy(zdeg_hbm, dstage_v)
        pltpu.sync_copy(dstage_v, deg_sh.at[pl.ds(sid * RPT, RPT)])
        pltpu.sync_copy(ones_hbm, ones_v)
    plsc.subcore_barrier()

    def gather(i, k):
        pltpu.async_copy(
            h_hbm.at[sidx_v.at[pl.ds(i * CHUNK, CHUNK)]], rows[k], sems[k])

    def gwait(k):
        pltpu.make_async_copy(
            h_hbm.at[sidx_v.at[pl.ds(0, CHUNK)]], rows[k], sems[k]).wait()

    def scatter(i, k):
        idx = didx_v.at[pl.ds(i * CHUNK, CHUNK)]
        pltpu.sync_copy(rows[k], agg_sh.at[idx], add=True)
        if with_deg:
            pltpu.sync_copy(ones_v, deg_sh.at[idx], add=True)

    gather(0, 0)
    gather(1, 1)

    def group(g, carry):
        j0 = 3 * g
        for c in range(3):
            j = j0 + c
            gwait(c)
            gather(j + 2, (c + 2) % 3)
            scatter(j, c)
        return carry

    lax.fori_loop(0, (NCHUNKS - 4) // 3, group, 0)
    gwait(0)
    gather(NCHUNKS - 2, 2)
    scatter(NCHUNKS - 4, 0)
    gwait(1)
    gather(NCHUNKS - 1, 0)
    scatter(NCHUNKS - 3, 1)
    gwait(2)
    scatter(NCHUNKS - 2, 2)
    gwait(0)
    scatter(NCHUNKS - 1, 0)
    plsc.subcore_barrier()

    for k in range(RCH):
        pltpu.sync_copy(agg_sh.at[pl.ds(sid * RPT + k * CHUNK, CHUNK)], rows0_v)
        pltpu.sync_copy(rows0_v, agg_out.at[cid, pl.ds(sid * RPT + k * CHUNK, CHUNK)])
    if with_deg:
        pltpu.sync_copy(deg_sh.at[pl.ds(sid * RPT, RPT)], dstage_v)
        pltpu.sync_copy(dstage_v, deg_out.at[pl.ds(cid * N_PAD + sid * RPT, RPT)])


def _sc_aggregate(h, src, dst, with_deg):
    mesh = plsc.VectorSubcoreMesh(core_axis_name="c", subcore_axis_name="s",
                                  num_cores=NC, num_subcores=NS)
    zfeat = jnp.zeros((CHUNK, D_FEAT), jnp.float32)
    if with_deg:
        out_type = (jax.ShapeDtypeStruct((NC, N_PAD, D_FEAT), jnp.float32),
                    jax.ShapeDtypeStruct((NC * N_PAD,), jnp.float32))
        scratch = [
            pltpu.VMEM_SHARED((N_PAD, D_FEAT), jnp.float32),
            pltpu.VMEM_SHARED((N_PAD,), jnp.float32),
            pltpu.VMEM((CHUNK, D_FEAT), jnp.float32),
            pltpu.VMEM((CHUNK, D_FEAT), jnp.float32),
            pltpu.VMEM((CHUNK, D_FEAT), jnp.float32),
            pltpu.VMEM((EPW,), jnp.int32),
            pltpu.VMEM((EPW,), jnp.int32),
            pltpu.VMEM((CHUNK,), jnp.float32),
            pltpu.VMEM((RPT,), jnp.float32),
            pltpu.SemaphoreType.DMA,
            pltpu.SemaphoreType.DMA,
            pltpu.SemaphoreType.DMA,
        ]
        zdeg = jnp.zeros((RPT,), jnp.float32)
        ones = jnp.ones((CHUNK,), jnp.float32)
        fn = pl.kernel(functools.partial(_sc_agg_body, True),
                       out_type=out_type, mesh=mesh, scratch_types=scratch)
        return fn(h, src, dst, zfeat, zdeg, ones)
    else:
        out_type = jax.ShapeDtypeStruct((NC, N_PAD, D_FEAT), jnp.float32)
        scratch = [
            pltpu.VMEM_SHARED((N_PAD, D_FEAT), jnp.float32),
            pltpu.VMEM((CHUNK, D_FEAT), jnp.float32),
            pltpu.VMEM((CHUNK, D_FEAT), jnp.float32),
            pltpu.VMEM((CHUNK, D_FEAT), jnp.float32),
            pltpu.VMEM((EPW,), jnp.int32),
            pltpu.VMEM((EPW,), jnp.int32),
            pltpu.SemaphoreType.DMA,
            pltpu.SemaphoreType.DMA,
            pltpu.SemaphoreType.DMA,
        ]
        fn = pl.kernel(functools.partial(_sc_agg_body, False),
                       out_type=out_type, mesh=mesh, scratch_types=scratch)
        return fn(h, src, dst, zfeat)


def _dense1_body(x_ref, agg_ref, degA_ref, degB_ref, Wr_ref, Wa_ref, b_ref, g_ref,
                 be_ref, o_ref):
    agg = agg_ref[0, :N_NODES] + agg_ref[1, :N_NODES]
    deg = degA_ref[:N_NODES] + degB_ref[:N_NODES]
    mean = agg / jnp.maximum(deg, 1.0)
    c = (jnp.dot(x_ref[...], Wr_ref[...], preferred_element_type=jnp.float32)
         + jnp.dot(mean, Wa_ref[...], preferred_element_type=jnp.float32)
         + b_ref[...])
    mu = jnp.mean(c, axis=0, keepdims=True)
    var = jnp.mean((c - mu) * (c - mu), axis=0, keepdims=True)
    h = (c - mu) * lax.rsqrt(var + 1e-5) * g_ref[...] + be_ref[...]
    o_ref[...] = jnp.maximum(h, 0.0)


def _dense1(x, agg, degA, degB, Wr, Wa, b, g, be):
    return pl.pallas_call(
        _dense1_body,
        out_shape=jax.ShapeDtypeStruct((N_NODES, D_FEAT), jnp.float32),
    )(x, agg, degA, degB, Wr, Wa, b.reshape(1, -1),
      g.reshape(1, -1), be.reshape(1, -1))


def _dense2_body(h_ref, agg_ref, degA_ref, degB_ref, Wr_ref, Wa_ref, b_ref, g_ref,
                 be_ref, Ws1_ref, bs1_ref, Ws2_ref, bs2_ref, Wh1_ref,
                 bh1_ref, Wh2_ref, bh2_ref, Wh3_ref, bh3_ref, o_ref):
    agg = agg_ref[0, :N_NODES] + agg_ref[1, :N_NODES]
    deg = degA_ref[:N_NODES] + degB_ref[:N_NODES]
    mean = agg / jnp.maximum(deg, 1.0)
    c = (jnp.dot(h_ref[...], Wr_ref[...], preferred_element_type=jnp.float32)
         + jnp.dot(mean, Wa_ref[...], preferred_element_type=jnp.float32)
         + b_ref[...])
    mu = jnp.mean(c, axis=0, keepdims=True)
    var = jnp.mean((c - mu) * (c - mu), axis=0, keepdims=True)
    h = (c - mu) * lax.rsqrt(var + 1e-5) * g_ref[...] + be_ref[...]
    h = jnp.maximum(h, 0.0)

    hg = jnp.mean(h.reshape(BATCH_SIZE, NODES_PER_GRAPH, D_FEAT), axis=1)

    t = jnp.maximum(hg, 0.0)
    t = jnp.dot(t, Ws1_ref[...], preferred_element_type=jnp.float32) + bs1_ref[...]
    t = jnp.dot(t, Ws2_ref[...], preferred_element_type=jnp.float32) + bs2_ref[...]
    t = jnp.maximum(t, 0.0)
    t = jnp.maximum(jnp.dot(t, Wh1_ref[...], preferred_element_type=jnp.float32) + bh1_ref[...], 0.0)
    t = jnp.maximum(jnp.dot(t, Wh2_ref[...], preferred_element_type=jnp.float32) + bh2_ref[...], 0.0)
    o_ref[...] = jnp.dot(t, Wh3_ref[...], preferred_element_type=jnp.float32) + bh3_ref[...]


def _dense2(h, agg, degA, degB, Wr, Wa, b, g, be, Ws1, bs1, Ws2, bs2,
            Wh1, bh1, Wh2, bh2, Wh3, bh3):
    return pl.pallas_call(
        _dense2_body,
        out_shape=jax.ShapeDtypeStruct((BATCH_SIZE, 1), jnp.float32),
    )(h, agg, degA, degB, Wr, Wa, b.reshape(1, -1), g.reshape(1, -1),
      be.reshape(1, -1), Ws1, bs1.reshape(1, -1), Ws2, bs2.reshape(1, -1),
      Wh1, bh1.reshape(1, -1), Wh2, bh2.reshape(1, -1), Wh3,
      bh3.reshape(1, -1))


def kernel(x, edge_index, batch, Wr0, Wa0, b0, g0, be0, Wr1, Wa1, b1, g1,
           be1, Ws1, bs1, Ws2, bs2, Wh1, bh1, Wh2, bh2, Wh3, bh3):
    npad = E_PAD - N_EDGES
    pad_dst = N_NODES + (jnp.arange(npad, dtype=jnp.int32)
                         % (N_PAD - N_NODES))
    src = jnp.concatenate([edge_index[0], jnp.zeros((npad,), jnp.int32)])
    dst = jnp.concatenate([edge_index[1], pad_dst])
    agg0, deg_flat = _sc_aggregate(x, src, dst, with_deg=True)
    degA = deg_flat[:N_PAD].reshape(N_PAD, 1)
    degB = deg_flat[N_PAD:].reshape(N_PAD, 1)
    h1 = _dense1(x, agg0, degA, degB, Wr0, Wa0, b0, g0, be0)
    agg1 = _sc_aggregate(h1, src, dst, with_deg=False)
    return _dense2(h1, agg1, degA, degB, Wr1, Wa1, b1, g1, be1, Ws1, bs1,
                   Ws2, bs2, Wh1, bh1, Wh2, bh2, Wh3, bh3)


# speedup vs baseline: 3.7217x; 3.2390x over previous
import functools

import jax
import jax.numpy as jnp
from jax import lax
from jax.experimental import pallas as pl
from jax.experimental.pallas import tpu as pltpu
from jax.experimental.pallas import tpu_sc as plsc

N_NODES = 10000
N_EDGES = 320000
D_FEAT = 128
BATCH_SIZE = 100
NODES_PER_GRAPH = 100

NC = 2
NS = 16
NW = NC * NS
CHUNK = 64
NCHUNKS = 160
EPW = NCHUNKS * CHUNK
E_PAD = NW * EPW
N_PAD = 10240
RPT = N_PAD // NS
RCH = RPT // CHUNK


def _sc_agg_body(with_deg, *refs):
    if with_deg:
        (h_hbm, src_hbm, dst_hbm, zfeat_hbm, zdeg_hbm, ones_hbm,
         agg_out, deg_out,
         agg_sh, deg_sh, rows0_v, rows1_v, rows2_v, sidx_v, didx_v,
         ones_v, dstage_v, sem0, sem1, sem2) = refs
    else:
        (h_hbm, src_hbm, dst_hbm, zfeat_hbm,
         agg_out,
         agg_sh, rows0_v, rows1_v, rows2_v, sidx_v, didx_v,
         sem0, sem1, sem2) = refs
    rows = (rows0_v, rows1_v, rows2_v)
    sems = (sem0, sem1, sem2)

    cid = lax.axis_index("c")
    sid = lax.axis_index("s")
    wid = sid * NC + cid

    pltpu.sync_copy(src_hbm.at[pl.ds(wid * EPW, EPW)], sidx_v)
    pltpu.sync_copy(dst_hbm.at[pl.ds(wid * EPW, EPW)], didx_v)

    pltpu.sync_copy(zfeat_hbm, rows0_v)
    for k in range(RCH):
        pltpu.sync_copy(rows0_v, agg_sh.at[pl.ds(sid * RPT + k * CHUNK, CHUNK)])
    if with_deg:
        pltpu.sync_copy(zdeg_hbm, dstage_v)
        pltpu.sync_copy(dstage_v, deg_sh.at[pl.ds(sid * RPT, RPT)])
        pltpu.sync_copy(ones_hbm, ones_v)
    plsc.subcore_barrier()

    def gather(i, k):
        pltpu.async_copy(
            h_hbm.at[sidx_v.at[pl.ds(i * CHUNK, CHUNK)]], rows[k], sems[k])

    def gwait(k):
        pltpu.make_async_copy(
            h_hbm.at[sidx_v.at[pl.ds(0, CHUNK)]], rows[k], sems[k]).wait()

    def scatter(i, k):
        idx = didx_v.at[pl.ds(i * CHUNK, CHUNK)]
        pltpu.sync_copy(rows[k], agg_sh.at[idx], add=True)
        if with_deg:
            pltpu.sync_copy(ones_v, deg_sh.at[idx], add=True)

    gather(0, 0)
    gather(1, 1)

    def group(g, carry):
        j0 = 3 * g
        for c in range(3):
            j = j0 + c
            gwait(c)
            gather(j + 2, (c + 2) % 3)
            scatter(j, c)
        return carry

    lax.fori_loop(0, (NCHUNKS - 4) // 3, group, 0)
    gwait(0)
    gather(NCHUNKS - 2, 2)
    scatter(NCHUNKS - 4, 0)
    gwait(1)
    gather(NCHUNKS - 1, 0)
    scatter(NCHUNKS - 3, 1)
    gwait(2)
    scatter(NCHUNKS - 2, 2)
    gwait(0)
    scatter(NCHUNKS - 1, 0)
    plsc.subcore_barrier()

    for k in range(RCH):
        pltpu.sync_copy(agg_sh.at[pl.ds(sid * RPT + k * CHUNK, CHUNK)], rows0_v)
        pltpu.sync_copy(rows0_v, agg_out.at[cid, pl.ds(sid * RPT + k * CHUNK, CHUNK)])
    if with_deg:
        pltpu.sync_copy(deg_sh.at[pl.ds(sid * RPT, RPT)], dstage_v)
        pltpu.sync_copy(dstage_v, deg_out.at[pl.ds(cid * N_PAD + sid * RPT, RPT)])


def _sc_aggregate(h, src, dst, with_deg):
    mesh = plsc.VectorSubcoreMesh(core_axis_name="c", subcore_axis_name="s",
                                  num_cores=NC, num_subcores=NS)
    zfeat = jnp.zeros((CHUNK, D_FEAT), jnp.float32)
    if with_deg:
        out_type = (jax.ShapeDtypeStruct((NC, N_PAD, D_FEAT), jnp.float32),
                    jax.ShapeDtypeStruct((NC * N_PAD,), jnp.float32))
        scratch = [
            pltpu.VMEM_SHARED((N_PAD, D_FEAT), jnp.float32),
            pltpu.VMEM_SHARED((N_PAD,), jnp.float32),
            pltpu.VMEM((CHUNK, D_FEAT), jnp.float32),
            pltpu.VMEM((CHUNK, D_FEAT), jnp.float32),
            pltpu.VMEM((CHUNK, D_FEAT), jnp.float32),
            pltpu.VMEM((EPW,), jnp.int32),
            pltpu.VMEM((EPW,), jnp.int32),
            pltpu.VMEM((CHUNK,), jnp.float32),
            pltpu.VMEM((RPT,), jnp.float32),
            pltpu.SemaphoreType.DMA,
            pltpu.SemaphoreType.DMA,
            pltpu.SemaphoreType.DMA,
        ]
        zdeg = jnp.zeros((RPT,), jnp.float32)
        ones = jnp.ones((CHUNK,), jnp.float32)
        fn = pl.kernel(functools.partial(_sc_agg_body, True),
                       out_type=out_type, mesh=mesh, scratch_types=scratch)
        return fn(h, src, dst, zfeat, zdeg, ones)
    else:
        out_type = jax.ShapeDtypeStruct((NC, N_PAD, D_FEAT), jnp.float32)
        scratch = [
            pltpu.VMEM_SHARED((N_PAD, D_FEAT), jnp.float32),
            pltpu.VMEM((CHUNK, D_FEAT), jnp.float32),
            pltpu.VMEM((CHUNK, D_FEAT), jnp.float32),
            pltpu.VMEM((CHUNK, D_FEAT), jnp.float32),
            pltpu.VMEM((EPW,), jnp.int32),
            pltpu.VMEM((EPW,), jnp.int32),
            pltpu.SemaphoreType.DMA,
            pltpu.SemaphoreType.DMA,
            pltpu.SemaphoreType.DMA,
        ]
        fn = pl.kernel(functools.partial(_sc_agg_body, False),
                       out_type=out_type, mesh=mesh, scratch_types=scratch)
        return fn(h, src, dst, zfeat)


def _dense1_body(x_ref, agg_ref, degA_ref, degB_ref, Wr_ref, Wa_ref, b_ref, g_ref,
                 be_ref, o_ref):
    agg = agg_ref[0, :N_NODES] + agg_ref[1, :N_NODES]
    deg = degA_ref[:N_NODES] + degB_ref[:N_NODES]
    mean = agg / jnp.maximum(deg, 1.0)
    c = (jnp.dot(x_ref[...], Wr_ref[...], preferred_element_type=jnp.float32)
         + jnp.dot(mean, Wa_ref[...], preferred_element_type=jnp.float32)
         + b_ref[...])
    mu = jnp.mean(c, axis=0, keepdims=True)
    var = jnp.mean((c - mu) * (c - mu), axis=0, keepdims=True)
    h = (c - mu) * lax.rsqrt(var + 1e-5) * g_ref[...] + be_ref[...]
    o_ref[...] = jnp.maximum(h, 0.0)


def _dense1(x, agg, degA, degB, Wr, Wa, b, g, be):
    return pl.pallas_call(
        _dense1_body,
        out_shape=jax.ShapeDtypeStruct((N_NODES, D_FEAT), jnp.float32),
    )(x, agg, degA, degB, Wr, Wa, b.reshape(1, -1),
      g.reshape(1, -1), be.reshape(1, -1))


def _dense2_body(h_ref, agg_ref, degA_ref, degB_ref, Wr_ref, Wa_ref, b_ref, g_ref,
                 be_ref, Ws1_ref, bs1_ref, Ws2_ref, bs2_ref, Wh1_ref,
                 bh1_ref, Wh2_ref, bh2_ref, Wh3_ref, bh3_ref, o_ref):
    agg = agg_ref[0, :N_NODES] + agg_ref[1, :N_NODES]
    deg = degA_ref[:N_NODES] + degB_ref[:N_NODES]
    mean = agg / jnp.maximum(deg, 1.0)
    c = (jnp.dot(h_ref[...], Wr_ref[...], preferred_element_type=jnp.float32)
         + jnp.dot(mean, Wa_ref[...], preferred_element_type=jnp.float32)
         + b_ref[...])
    mu = jnp.mean(c, axis=0, keepdims=True)
    var = jnp.mean((c - mu) * (c - mu), axis=0, keepdims=True)
    h = (c - mu) * lax.rsqrt(var + 1e-5) * g_ref[...] + be_ref[...]
    h = jnp.maximum(h, 0.0)

    hg = jnp.mean(h.reshape(BATCH_SIZE, NODES_PER_GRAPH, D_FEAT), axis=1)

    t = jnp.maximum(hg, 0.0)
    t = jnp.dot(t, Ws1_ref[...], preferred_element_type=jnp.float32) + bs1_ref[...]
    t = jnp.dot(t, Ws2_ref[...], preferred_element_type=jnp.float32) + bs2_ref[...]
    t = jnp.maximum(t, 0.0)
    t = jnp.maximum(jnp.dot(t, Wh1_ref[...], preferred_element_type=jnp.float32) + bh1_ref[...], 0.0)
    t = jnp.maximum(jnp.dot(t, Wh2_ref[...], preferred_element_type=jnp.float32) + bh2_ref[...], 0.0)
    o_ref[...] = jnp.dot(t, Wh3_ref[...], preferred_element_type=jnp.float32) + bh3_ref[...]


def _dense2(h, agg, degA, degB, Wr, Wa, b, g, be, Ws1, bs1, Ws2, bs2,
            Wh1, bh1, Wh2, bh2, Wh3, bh3):
    return pl.pallas_call(
        _dense2_body,
        out_shape=jax.ShapeDtypeStruct((BATCH_SIZE, 1), jnp.float32),
    )(h, agg, degA, degB, Wr, Wa, b.reshape(1, -1), g.reshape(1, -1),
      be.reshape(1, -1), Ws1, bs1.reshape(1, -1), Ws2, bs2.reshape(1, -1),
      Wh1, bh1.reshape(1, -1), Wh2, bh2.reshape(1, -1), Wh3,
      bh3.reshape(1, -1))


def kernel(x, edge_index, batch, Wr0, Wa0, b0, g0, be0, Wr1, Wa1, b1, g1,
           be1, Ws1, bs1, Ws2, bs2, Wh1, bh1, Wh2, bh2, Wh3, bh3):
    npad = E_PAD - N_EDGES
    pad_dst = N_NODES + (jnp.arange(npad, dtype=jnp.int32)
                         % (N_PAD - N_NODES))
    pad_src = jnp.arange(npad, dtype=jnp.int32) % N_NODES
    src = jnp.concatenate([edge_index[0], pad_src])
    dst = jnp.concatenate([edge_index[1], pad_dst])
    agg0, deg_flat = _sc_aggregate(x, src, dst, with_deg=True)
    degA = deg_flat[:N_PAD].reshape(N_PAD, 1)
    degB = deg_flat[N_PAD:].reshape(N_PAD, 1)
    h1 = _dense1(x, agg0, degA, degB, Wr0, Wa0, b0, g0, be0)
    agg1 = _sc_aggregate(h1, src, dst, with_deg=False)
    return _dense2(h1, agg1, degA, degB, Wr1, Wa1, b1, g1, be1, Ws1, bs1,
                   Ws2, bs2, Wh1, bh1, Wh2, bh2, Wh3, bh3)


# speedup vs baseline: 3.8366x; 1.0309x over previous
import functools

import jax
import jax.numpy as jnp
from jax import lax
from jax.experimental import pallas as pl
from jax.experimental.pallas import tpu as pltpu
from jax.experimental.pallas import tpu_sc as plsc

N_NODES = 10000
N_EDGES = 320000
D_FEAT = 128
BATCH_SIZE = 100
NODES_PER_GRAPH = 100

NC = 2
NS = 16
NW = NC * NS
CHUNK = 64
NCHUNKS = 160
EPW = NCHUNKS * CHUNK
E_PAD = NW * EPW
N_PAD = 10240
RPT = N_PAD // NS
RCH = RPT // CHUNK


def _sc_agg_body(with_deg, *refs):
    if with_deg:
        (h_hbm, src_hbm, dst_hbm, zfeat_hbm, zdeg_hbm, ones_hbm,
         agg_out, deg_out,
         agg_sh, deg_sh, rows0_v, rows1_v, rows2_v, sidx_v, didx_v,
         ones_v, dstage_v, sem0, sem1, sem2) = refs
    else:
        (h_hbm, src_hbm, dst_hbm, zfeat_hbm,
         agg_out,
         agg_sh, rows0_v, rows1_v, rows2_v, sidx_v, didx_v,
         sem0, sem1, sem2) = refs
    rows = (rows0_v, rows1_v, rows2_v)
    sems = (sem0, sem1, sem2)

    cid = lax.axis_index("c")
    sid = lax.axis_index("s")
    wid = sid * NC + cid

    pltpu.async_copy(src_hbm.at[pl.ds(wid * EPW, EPW)], sidx_v, sem0)
    pltpu.async_copy(dst_hbm.at[pl.ds(wid * EPW, EPW)], didx_v, sem1)

    pltpu.sync_copy(zfeat_hbm, rows0_v)
    for k in range(RCH):
        pltpu.sync_copy(rows0_v, agg_sh.at[pl.ds(sid * RPT + k * CHUNK, CHUNK)])
    if with_deg:
        pltpu.sync_copy(zdeg_hbm, dstage_v)
        pltpu.sync_copy(dstage_v, deg_sh.at[pl.ds(sid * RPT, RPT)])
        pltpu.sync_copy(ones_hbm, ones_v)
    pltpu.make_async_copy(
        src_hbm.at[pl.ds(wid * EPW, EPW)], sidx_v, sem0).wait()
    pltpu.make_async_copy(
        dst_hbm.at[pl.ds(wid * EPW, EPW)], didx_v, sem1).wait()

    def gather(i, k):
        pltpu.async_copy(
            h_hbm.at[sidx_v.at[pl.ds(i * CHUNK, CHUNK)]], rows[k], sems[k])

    def gwait(k):
        pltpu.make_async_copy(
            h_hbm.at[sidx_v.at[pl.ds(0, CHUNK)]], rows[k], sems[k]).wait()

    def scatter(i, k):
        idx = didx_v.at[pl.ds(i * CHUNK, CHUNK)]
        pltpu.sync_copy(rows[k], agg_sh.at[idx], add=True)
        if with_deg:
            pltpu.sync_copy(ones_v, deg_sh.at[idx], add=True)

    gather(0, 0)
    gather(1, 1)
    plsc.subcore_barrier()

    def group(g, carry):
        j0 = 3 * g
        for c in range(3):
            j = j0 + c
            gwait(c)
            gather(j + 2, (c + 2) % 3)
            scatter(j, c)
        return carry

    lax.fori_loop(0, (NCHUNKS - 4) // 3, group, 0)
    gwait(0)
    gather(NCHUNKS - 2, 2)
    scatter(NCHUNKS - 4, 0)
    gwait(1)
    gather(NCHUNKS - 1, 0)
    scatter(NCHUNKS - 3, 1)
    gwait(2)
    scatter(NCHUNKS - 2, 2)
    gwait(0)
    scatter(NCHUNKS - 1, 0)
    plsc.subcore_barrier()

    for k in range(3):
        pltpu.async_copy(
            agg_sh.at[pl.ds(sid * RPT + k * CHUNK, CHUNK)], rows[k], sems[k])
    for k in range(RCH):
        m = k % 3
        pltpu.make_async_copy(
            agg_sh.at[pl.ds(sid * RPT, CHUNK)], rows[m], sems[m]).wait()
        pltpu.sync_copy(rows[m], agg_out.at[cid, pl.ds(sid * RPT + k * CHUNK, CHUNK)])
        if k + 3 < RCH:
            pltpu.async_copy(
                agg_sh.at[pl.ds(sid * RPT + (k + 3) * CHUNK, CHUNK)],
                rows[m], sems[m])
    if with_deg:
        pltpu.sync_copy(deg_sh.at[pl.ds(sid * RPT, RPT)], dstage_v)
        pltpu.sync_copy(dstage_v, deg_out.at[pl.ds(cid * N_PAD + sid * RPT, RPT)])


def _sc_aggregate(h, src, dst, with_deg):
    mesh = plsc.VectorSubcoreMesh(core_axis_name="c", subcore_axis_name="s",
                                  num_cores=NC, num_subcores=NS)
    zfeat = jnp.zeros((CHUNK, D_FEAT), jnp.float32)
    if with_deg:
        out_type = (jax.ShapeDtypeStruct((NC, N_PAD, D_FEAT), jnp.float32),
                    jax.ShapeDtypeStruct((NC * N_PAD,), jnp.float32))
        scratch = [
            pltpu.VMEM_SHARED((N_PAD, D_FEAT), jnp.float32),
            pltpu.VMEM_SHARED((N_PAD,), jnp.float32),
            pltpu.VMEM((CHUNK, D_FEAT), jnp.float32),
            pltpu.VMEM((CHUNK, D_FEAT), jnp.float32),
            pltpu.VMEM((CHUNK, D_FEAT), jnp.float32),
            pltpu.VMEM((EPW,), jnp.int32),
            pltpu.VMEM((EPW,), jnp.int32),
            pltpu.VMEM((CHUNK,), jnp.float32),
            pltpu.VMEM((RPT,), jnp.float32),
            pltpu.SemaphoreType.DMA,
            pltpu.SemaphoreType.DMA,
            pltpu.SemaphoreType.DMA,
        ]
        zdeg = jnp.zeros((RPT,), jnp.float32)
        ones = jnp.ones((CHUNK,), jnp.float32)
        fn = pl.kernel(functools.partial(_sc_agg_body, True),
                       out_type=out_type, mesh=mesh, scratch_types=scratch)
        return fn(h, src, dst, zfeat, zdeg, ones)
    else:
        out_type = jax.ShapeDtypeStruct((NC, N_PAD, D_FEAT), jnp.float32)
        scratch = [
            pltpu.VMEM_SHARED((N_PAD, D_FEAT), jnp.float32),
            pltpu.VMEM((CHUNK, D_FEAT), jnp.float32),
            pltpu.VMEM((CHUNK, D_FEAT), jnp.float32),
            pltpu.VMEM((CHUNK, D_FEAT), jnp.float32),
            pltpu.VMEM((EPW,), jnp.int32),
            pltpu.VMEM((EPW,), jnp.int32),
            pltpu.SemaphoreType.DMA,
            pltpu.SemaphoreType.DMA,
            pltpu.SemaphoreType.DMA,
        ]
        fn = pl.kernel(functools.partial(_sc_agg_body, False),
                       out_type=out_type, mesh=mesh, scratch_types=scratch)
        return fn(h, src, dst, zfeat)


def _dense1_body(x_ref, agg_ref, degA_ref, degB_ref, Wr_ref, Wa_ref, b_ref, g_ref,
                 be_ref, o_ref):
    agg = agg_ref[0, :N_NODES] + agg_ref[1, :N_NODES]
    deg = degA_ref[:N_NODES] + degB_ref[:N_NODES]
    mean = agg / jnp.maximum(deg, 1.0)
    c = (jnp.dot(x_ref[...], Wr_ref[...], preferred_element_type=jnp.float32)
         + jnp.dot(mean, Wa_ref[...], preferred_element_type=jnp.float32)
         + b_ref[...])
    mu = jnp.mean(c, axis=0, keepdims=True)
    var = jnp.mean((c - mu) * (c - mu), axis=0, keepdims=True)
    h = (c - mu) * lax.rsqrt(var + 1e-5) * g_ref[...] + be_ref[...]
    o_ref[...] = jnp.maximum(h, 0.0)


def _dense1(x, agg, degA, degB, Wr, Wa, b, g, be):
    return pl.pallas_call(
        _dense1_body,
        out_shape=jax.ShapeDtypeStruct((N_NODES, D_FEAT), jnp.float32),
    )(x, agg, degA, degB, Wr, Wa, b.reshape(1, -1),
      g.reshape(1, -1), be.reshape(1, -1))


def _dense2_body(h_ref, agg_ref, degA_ref, degB_ref, Wr_ref, Wa_ref, b_ref, g_ref,
                 be_ref, Ws1_ref, bs1_ref, Ws2_ref, bs2_ref, Wh1_ref,
                 bh1_ref, Wh2_ref, bh2_ref, Wh3_ref, bh3_ref, o_ref):
    agg = agg_ref[0, :N_NODES] + agg_ref[1, :N_NODES]
    deg = degA_ref[:N_NODES] + degB_ref[:N_NODES]
    mean = agg / jnp.maximum(deg, 1.0)
    c = (jnp.dot(h_ref[...], Wr_ref[...], preferred_element_type=jnp.float32)
         + jnp.dot(mean, Wa_ref[...], preferred_element_type=jnp.float32)
         + b_ref[...])
    mu = jnp.mean(c, axis=0, keepdims=True)
    var = jnp.mean((c - mu) * (c - mu), axis=0, keepdims=True)
    h = (c - mu) * lax.rsqrt(var + 1e-5) * g_ref[...] + be_ref[...]
    h = jnp.maximum(h, 0.0)

    hg = jnp.mean(h.reshape(BATCH_SIZE, NODES_PER_GRAPH, D_FEAT), axis=1)

    t = jnp.maximum(hg, 0.0)
    t = jnp.dot(t, Ws1_ref[...], preferred_element_type=jnp.float32) + bs1_ref[...]
    t = jnp.dot(t, Ws2_ref[...], preferred_element_type=jnp.float32) + bs2_ref[...]
    t = jnp.maximum(t, 0.0)
    t = jnp.maximum(jnp.dot(t, Wh1_ref[...], preferred_element_type=jnp.float32) + bh1_ref[...], 0.0)
    t = jnp.maximum(jnp.dot(t, Wh2_ref[...], preferred_element_type=jnp.float32) + bh2_ref[...], 0.0)
    o_ref[...] = jnp.dot(t, Wh3_ref[...], preferred_element_type=jnp.float32) + bh3_ref[...]


def _dense2(h, agg, degA, degB, Wr, Wa, b, g, be, Ws1, bs1, Ws2, bs2,
            Wh1, bh1, Wh2, bh2, Wh3, bh3):
    return pl.pallas_call(
        _dense2_body,
        out_shape=jax.ShapeDtypeStruct((BATCH_SIZE, 1), jnp.float32),
    )(h, agg, degA, degB, Wr, Wa, b.reshape(1, -1), g.reshape(1, -1),
      be.reshape(1, -1), Ws1, bs1.reshape(1, -1), Ws2, bs2.reshape(1, -1),
      Wh1, bh1.reshape(1, -1), Wh2, bh2.reshape(1, -1), Wh3,
      bh3.reshape(1, -1))


def kernel(x, edge_index, batch, Wr0, Wa0, b0, g0, be0, Wr1, Wa1, b1, g1,
           be1, Ws1, bs1, Ws2, bs2, Wh1, bh1, Wh2, bh2, Wh3, bh3):
    npad = E_PAD - N_EDGES
    pad_dst = N_NODES + (jnp.arange(npad, dtype=jnp.int32)
                         % (N_PAD - N_NODES))
    pad_src = jnp.arange(npad, dtype=jnp.int32) % N_NODES
    src = jnp.concatenate([edge_index[0], pad_src])
    dst = jnp.concatenate([edge_index[1], pad_dst])
    agg0, deg_flat = _sc_aggregate(x, src, dst, with_deg=True)
    degA = deg_flat[:N_PAD].reshape(N_PAD, 1)
    degB = deg_flat[N_PAD:].reshape(N_PAD, 1)
    h1 = _dense1(x, agg0, degA, degB, Wr0, Wa0, b0, g0, be0)
    agg1 = _sc_aggregate(h1, src, dst, with_deg=False)
    return _dense2(h1, agg1, degA, degB, Wr1, Wa1, b1, g1, be1, Ws1, bs1,
                   Ws2, bs2, Wh1, bh1, Wh2, bh2, Wh3, bh3)
